# baseline (speedup 1.0000x reference)
import functools

import jax
import jax.numpy as jnp
from jax import lax
from jax.experimental import pallas as pl
from jax.experimental.pallas import tpu as pltpu
from jax.experimental.pallas import tpu_sc as plsc

_EPS = 0.001
_LANES = 16


def _sc_make(num_rows, chunk, batch, idx_pad):
    info = plsc.get_sparse_core_info()
    nc, ns = info.num_cores, info.num_subcores
    nw = nc * ns
    mesh = plsc.VectorSubcoreMesh(core_axis_name="c", subcore_axis_name="s")

    @functools.partial(
        pl.kernel,
        out_type=(
            jax.ShapeDtypeStruct((nw, batch, chunk), jnp.float32),
            jax.ShapeDtypeStruct((nw, batch, chunk), jnp.float32),
        ),
        mesh=mesh,
        scratch_types=[
            pltpu.VMEM((idx_pad,), jnp.int32),
            pltpu.VMEM((idx_pad, chunk), jnp.float32),
            pltpu.VMEM((idx_pad, chunk), jnp.float32),
            pltpu.VMEM((batch, chunk), jnp.float32),
            pltpu.VMEM((batch, chunk), jnp.float32),
            pltpu.SemaphoreType.DMA,
            pltpu.SemaphoreType.DMA,
        ],
    )
    def sc_kernel(idx_hbm, mean_hbm, std_hbm, scale_hbm, off_hbm,
                  idx_v, mean_v, std_v, scale_v, off_v, sem0, sem1):
        wid = lax.axis_index("s") * nc + lax.axis_index("c")
        pltpu.sync_copy(idx_hbm.at[wid], idx_v)
        cp_m = pltpu.async_copy(mean_hbm.at[idx_v], mean_v, sem0)
        cp_s = pltpu.async_copy(std_hbm.at[idx_v], std_v, sem1)
        cp_m.wait()
        cp_s.wait()
        for b in range(batch):
            for i in range(chunk // _LANES):
                sl = pl.ds(i * _LANES, _LANES)
                s = std_v[b, sl]
                m = mean_v[b, sl]
                sc = 1.0 / (jnp.exp(s) + _EPS)
                scale_v[b, sl] = sc
                off_v[b, sl] = -m * sc
        pltpu.sync_copy(scale_v, scale_hbm.at[wid])
        pltpu.sync_copy(off_v, off_hbm.at[wid])

    return sc_kernel


def _tc_body(x_ref, scale_ref, off_ref, o_ref):
    o_ref[...] = x_ref[...] * scale_ref[...] + off_ref[...]


def kernel(x, context_id, initial_mean, initial_std):
    b, s, d = x.shape
    num_ctx = initial_mean.shape[0]
    info = plsc.get_sparse_core_info()
    nw = info.num_cores * info.num_subcores
    chunk = d // nw

    cid = context_id[:, 0].astype(jnp.int32)
    idx = cid[None, :] * nw + jnp.arange(nw, dtype=jnp.int32)[:, None]
    idx_pad = 8 * ((b + 7) // 8)
    reps = idx_pad // b + (1 if idx_pad % b else 0)
    idx8 = jnp.tile(idx, (1, reps))[:, :idx_pad]

    mean_r = initial_mean.reshape(num_ctx * nw, chunk)
    std_r = initial_std.reshape(num_ctx * nw, chunk)

    scale_w, off_w = _sc_make(num_ctx * nw, chunk, b, idx_pad)(
        idx8, mean_r, std_r)
    scale = scale_w.transpose(1, 0, 2).reshape(b, 1, d)
    off = off_w.transpose(1, 0, 2).reshape(b, 1, d)

    bs = 512
    grid = (b, s // bs)
    out = pl.pallas_call(
        _tc_body,
        grid=grid,
        in_specs=[
            pl.BlockSpec((1, bs, d), lambda i, j: (i, j, 0)),
            pl.BlockSpec((1, 1, d), lambda i, j: (i, 0, 0)),
            pl.BlockSpec((1, 1, d), lambda i, j: (i, 0, 0)),
        ],
        out_specs=pl.BlockSpec((1, bs, d), lambda i, j: (i, j, 0)),
        out_shape=jax.ShapeDtypeStruct((b, s, d), x.dtype),
        compiler_params=pltpu.CompilerParams(
            dimension_semantics=("parallel", "parallel"),
        ),
    )(x, scale, off)
    return out


# speedup vs baseline: 1.1185x; 1.1185x over previous
import functools

import jax
import jax.numpy as jnp
from jax import lax
from jax.experimental import pallas as pl
from jax.experimental.pallas import tpu as pltpu
from jax.experimental.pallas import tpu_sc as plsc

_EPS = 0.001
_LANES = 16


def _sc_make(num_rows, chunk, batch, idx_pad):
    info = plsc.get_sparse_core_info()
    nc, ns = info.num_cores, info.num_subcores
    nw = nc * ns
    mesh = plsc.VectorSubcoreMesh(core_axis_name="c", subcore_axis_name="s")

    @functools.partial(
        pl.kernel,
        out_type=(
            jax.ShapeDtypeStruct((nw, batch, chunk), jnp.float32),
            jax.ShapeDtypeStruct((nw, batch, chunk), jnp.float32),
        ),
        mesh=mesh,
        scratch_types=[
            pltpu.VMEM((idx_pad,), jnp.int32),
            pltpu.VMEM((idx_pad, chunk), jnp.float32),
            pltpu.VMEM((idx_pad, chunk), jnp.float32),
            pltpu.VMEM((batch, chunk), jnp.float32),
            pltpu.VMEM((batch, chunk), jnp.float32),
            pltpu.SemaphoreType.DMA,
            pltpu.SemaphoreType.DMA,
        ],
    )
    def sc_kernel(idx_hbm, mean_hbm, std_hbm, scale_hbm, off_hbm,
                  idx_v, mean_v, std_v, scale_v, off_v, sem0, sem1):
        wid = lax.axis_index("s") * nc + lax.axis_index("c")
        pltpu.sync_copy(idx_hbm.at[wid], idx_v)
        cp_m = pltpu.async_copy(mean_hbm.at[idx_v], mean_v, sem0)
        cp_s = pltpu.async_copy(std_hbm.at[idx_v], std_v, sem1)
        cp_m.wait()
        cp_s.wait()
        for b in range(batch):
            for i in range(chunk // _LANES):
                sl = pl.ds(i * _LANES, _LANES)
                s = std_v[b, sl]
                m = mean_v[b, sl]
                sc = 1.0 / (jnp.exp(s) + _EPS)
                scale_v[b, sl] = sc
                off_v[b, sl] = -m * sc
        pltpu.sync_copy(scale_v, scale_hbm.at[wid])
        pltpu.sync_copy(off_v, off_hbm.at[wid])

    return sc_kernel


def _tc_body(cid_ref, mean_ref, std_ref, x_ref, o_ref):
    sc = 1.0 / (jnp.exp(std_ref[...]) + _EPS)
    o_ref[...] = (x_ref[...] - mean_ref[...]) * sc


def kernel(x, context_id, initial_mean, initial_std):
    b, s, d = x.shape
    num_ctx = initial_mean.shape[0]
    cid = context_id[:, 0].astype(jnp.int32)
    mean3 = initial_mean.reshape(num_ctx, 1, d)
    std3 = initial_std.reshape(num_ctx, 1, d)

    bs = 512
    grid = (b, s // bs)
    out = pl.pallas_call(
        _tc_body,
        grid_spec=pltpu.PrefetchScalarGridSpec(
            num_scalar_prefetch=1,
            grid=grid,
            in_specs=[
                pl.BlockSpec((1, 1, d), lambda i, j, cid_ref: (cid_ref[i], 0, 0)),
                pl.BlockSpec((1, 1, d), lambda i, j, cid_ref: (cid_ref[i], 0, 0)),
                pl.BlockSpec((1, bs, d), lambda i, j, cid_ref: (i, j, 0)),
            ],
            out_specs=pl.BlockSpec((1, bs, d), lambda i, j, cid_ref: (i, j, 0)),
        ),
        out_shape=jax.ShapeDtypeStruct((b, s, d), x.dtype),
        compiler_params=pltpu.CompilerParams(
            dimension_semantics=("arbitrary", "arbitrary"),
        ),
    )(cid, mean3, std3, x)
    return out


# speedup vs baseline: 1.1190x; 1.0004x over previous
import functools

import jax
import jax.numpy as jnp
from jax import lax
from jax.experimental import pallas as pl
from jax.experimental.pallas import tpu as pltpu
from jax.experimental.pallas import tpu_sc as plsc

_EPS = 0.001
_LANES = 16


def _sc_make(num_rows, chunk, batch, idx_pad):
    info = plsc.get_sparse_core_info()
    nc, ns = info.num_cores, info.num_subcores
    nw = nc * ns
    mesh = plsc.VectorSubcoreMesh(core_axis_name="c", subcore_axis_name="s")

    @functools.partial(
        pl.kernel,
        out_type=(
            jax.ShapeDtypeStruct((nw, batch, chunk), jnp.float32),
            jax.ShapeDtypeStruct((nw, batch, chunk), jnp.float32),
        ),
        mesh=mesh,
        scratch_types=[
            pltpu.VMEM((idx_pad,), jnp.int32),
            pltpu.VMEM((idx_pad, chunk), jnp.float32),
            pltpu.VMEM((idx_pad, chunk), jnp.float32),
            pltpu.VMEM((batch, chunk), jnp.float32),
            pltpu.VMEM((batch, chunk), jnp.float32),
            pltpu.SemaphoreType.DMA,
            pltpu.SemaphoreType.DMA,
        ],
    )
    def sc_kernel(idx_hbm, mean_hbm, std_hbm, scale_hbm, off_hbm,
                  idx_v, mean_v, std_v, scale_v, off_v, sem0, sem1):
        wid = lax.axis_index("s") * nc + lax.axis_index("c")
        pltpu.sync_copy(idx_hbm.at[wid], idx_v)
        cp_m = pltpu.async_copy(mean_hbm.at[idx_v], mean_v, sem0)
        cp_s = pltpu.async_copy(std_hbm.at[idx_v], std_v, sem1)
        cp_m.wait()
        cp_s.wait()
        for b in range(batch):
            for i in range(chunk // _LANES):
                sl = pl.ds(i * _LANES, _LANES)
                s = std_v[b, sl]
                m = mean_v[b, sl]
                sc = 1.0 / (jnp.exp(s) + _EPS)
                scale_v[b, sl] = sc
                off_v[b, sl] = -m * sc
        pltpu.sync_copy(scale_v, scale_hbm.at[wid])
        pltpu.sync_copy(off_v, off_hbm.at[wid])

    return sc_kernel


def _tc_body(cid_ref, mean_ref, std_ref, x_ref, o_ref, sc_ref, off_ref):
    @pl.when(pl.program_id(1) == 0)
    def _():
        sc = 1.0 / (jnp.exp(std_ref[...]) + _EPS)
        sc_ref[...] = sc
        off_ref[...] = -mean_ref[...] * sc

    o_ref[...] = x_ref[...] * sc_ref[...] + off_ref[...]


def kernel(x, context_id, initial_mean, initial_std):
    b, s, d = x.shape
    num_ctx = initial_mean.shape[0]
    cid = context_id[:, 0].astype(jnp.int32)
    mean3 = initial_mean.reshape(num_ctx, 1, d)
    std3 = initial_std.reshape(num_ctx, 1, d)

    bs = 512
    grid = (b, s // bs)
    out = pl.pallas_call(
        _tc_body,
        grid_spec=pltpu.PrefetchScalarGridSpec(
            num_scalar_prefetch=1,
            grid=grid,
            in_specs=[
                pl.BlockSpec((1, 1, d), lambda i, j, cid_ref: (cid_ref[i], 0, 0)),
                pl.BlockSpec((1, 1, d), lambda i, j, cid_ref: (cid_ref[i], 0, 0)),
                pl.BlockSpec((1, bs, d), lambda i, j, cid_ref: (i, j, 0)),
            ],
            out_specs=pl.BlockSpec((1, bs, d), lambda i, j, cid_ref: (i, j, 0)),
            scratch_shapes=[
                pltpu.VMEM((1, 1, d), jnp.float32),
                pltpu.VMEM((1, 1, d), jnp.float32),
            ],
        ),
        out_shape=jax.ShapeDtypeStruct((b, s, d), x.dtype),
        compiler_params=pltpu.CompilerParams(
            dimension_semantics=("arbitrary", "arbitrary"),
        ),
    )(cid, mean3, std3, x)
    return out


# speedup vs baseline: 1.1194x; 1.0003x over previous
import functools

import jax
import jax.numpy as jnp
from jax import lax
from jax.experimental import pallas as pl
from jax.experimental.pallas import tpu as pltpu
from jax.experimental.pallas import tpu_sc as plsc

_EPS = 0.001
_LANES = 16


def _sc_make(num_rows, chunk, batch, idx_pad):
    info = plsc.get_sparse_core_info()
    nc, ns = info.num_cores, info.num_subcores
    nw = nc * ns
    mesh = plsc.VectorSubcoreMesh(core_axis_name="c", subcore_axis_name="s")

    @functools.partial(
        pl.kernel,
        out_type=(
            jax.ShapeDtypeStruct((nw, batch, chunk), jnp.float32),
            jax.ShapeDtypeStruct((nw, batch, chunk), jnp.float32),
        ),
        mesh=mesh,
        scratch_types=[
            pltpu.VMEM((idx_pad,), jnp.int32),
            pltpu.VMEM((idx_pad, chunk), jnp.float32),
            pltpu.VMEM((idx_pad, chunk), jnp.float32),
            pltpu.VMEM((batch, chunk), jnp.float32),
            pltpu.VMEM((batch, chunk), jnp.float32),
            pltpu.SemaphoreType.DMA,
            pltpu.SemaphoreType.DMA,
        ],
    )
    def sc_kernel(idx_hbm, mean_hbm, std_hbm, scale_hbm, off_hbm,
                  idx_v, mean_v, std_v, scale_v, off_v, sem0, sem1):
        wid = lax.axis_index("s") * nc + lax.axis_index("c")
        pltpu.sync_copy(idx_hbm.at[wid], idx_v)
        cp_m = pltpu.async_copy(mean_hbm.at[idx_v], mean_v, sem0)
        cp_s = pltpu.async_copy(std_hbm.at[idx_v], std_v, sem1)
        cp_m.wait()
        cp_s.wait()
        for b in range(batch):
            for i in range(chunk // _LANES):
                sl = pl.ds(i * _LANES, _LANES)
                s = std_v[b, sl]
                m = mean_v[b, sl]
                sc = 1.0 / (jnp.exp(s) + _EPS)
                scale_v[b, sl] = sc
                off_v[b, sl] = -m * sc
        pltpu.sync_copy(scale_v, scale_hbm.at[wid])
        pltpu.sync_copy(off_v, off_hbm.at[wid])

    return sc_kernel


def _tc_body(cid_ref, mean_ref, std_ref, x_ref, o_ref, sc_ref, off_ref):
    @pl.when(pl.program_id(1) == 0)
    def _():
        sc = 1.0 / (jnp.exp(std_ref[...]) + _EPS)
        sc_ref[...] = sc
        off_ref[...] = -mean_ref[...] * sc

    o_ref[...] = x_ref[...] * sc_ref[...] + off_ref[...]


def kernel(x, context_id, initial_mean, initial_std):
    b, s, d = x.shape
    num_ctx = initial_mean.shape[0]
    cid = context_id[:, 0].astype(jnp.int32)
    mean3 = initial_mean.reshape(num_ctx, 1, d)
    std3 = initial_std.reshape(num_ctx, 1, d)

    bs = 512
    grid = (b, s // bs)
    out = pl.pallas_call(
        _tc_body,
        grid_spec=pltpu.PrefetchScalarGridSpec(
            num_scalar_prefetch=1,
            grid=grid,
            in_specs=[
                pl.BlockSpec((1, 1, d), lambda i, j, cid_ref: (cid_ref[i], 0, 0)),
                pl.BlockSpec((1, 1, d), lambda i, j, cid_ref: (cid_ref[i], 0, 0)),
                pl.BlockSpec((1, bs, d), lambda i, j, cid_ref: (i, j, 0)),
            ],
            out_specs=pl.BlockSpec((1, bs, d), lambda i, j, cid_ref: (i, j, 0)),
            scratch_shapes=[
                pltpu.VMEM((1, 1, d), jnp.float32),
                pltpu.VMEM((1, 1, d), jnp.float32),
            ],
        ),
        out_shape=jax.ShapeDtypeStruct((b, s, d), x.dtype),
        compiler_params=pltpu.CompilerParams(
            dimension_semantics=("parallel", "arbitrary"),
        ),
    )(cid, mean3, std3, x)
    return out


# speedup vs baseline: 1.1399x; 1.0183x over previous
import functools

import jax
import jax.numpy as jnp
from jax import lax
from jax.experimental import pallas as pl
from jax.experimental.pallas import tpu as pltpu
from jax.experimental.pallas import tpu_sc as plsc

_EPS = 0.001
_LANES = 16


def _tc_body(mean_ref, std_ref, x_ref, o_ref, sc_ref, off_ref):
    @pl.when(pl.program_id(1) == 0)
    def _():
        sc = 1.0 / (jnp.exp(std_ref[...]) + _EPS)
        sc_ref[...] = sc
        off_ref[...] = -mean_ref[...] * sc

    o_ref[...] = x_ref[...] * sc_ref[...] + off_ref[...]


def kernel(x, context_id, initial_mean, initial_std):
    b, s, d = x.shape
    cid = context_id[:, 0].astype(jnp.int32)
    mean_rows = jnp.take(initial_mean, cid, axis=0).reshape(b, 1, d)
    std_rows = jnp.take(initial_std, cid, axis=0).reshape(b, 1, d)

    bs = 512
    grid = (b, s // bs)
    out = pl.pallas_call(
        _tc_body,
        grid=grid,
        in_specs=[
            pl.BlockSpec((1, 1, d), lambda i, j: (i, 0, 0)),
            pl.BlockSpec((1, 1, d), lambda i, j: (i, 0, 0)),
            pl.BlockSpec((1, bs, d), lambda i, j: (i, j, 0)),
        ],
        out_specs=pl.BlockSpec((1, bs, d), lambda i, j: (i, j, 0)),
        out_shape=jax.ShapeDtypeStruct((b, s, d), x.dtype),
        scratch_shapes=[
            pltpu.VMEM((1, 1, d), jnp.float32),
            pltpu.VMEM((1, 1, d), jnp.float32),
        ],
        compiler_params=pltpu.CompilerParams(
            dimension_semantics=("parallel", "arbitrary"),
        ),
    )(mean_rows, std_rows, x)
    return out


# speedup vs baseline: 1.1649x; 1.0219x over previous
import functools

import jax
import jax.numpy as jnp
from jax import lax
from jax.experimental import pallas as pl
from jax.experimental.pallas import tpu as pltpu
from jax.experimental.pallas import tpu_sc as plsc

_EPS = 0.001


def _tc_body(cid_ref, mean_t_ref, std_t_ref, x_ref, o_ref, sc_ref, off_ref):
    @pl.when(pl.program_id(1) == 0)
    def _():
        c = cid_ref[pl.program_id(0)]
        srow = std_t_ref[pl.ds(c, 1), :]
        mrow = mean_t_ref[pl.ds(c, 1), :]
        sc = 1.0 / (jnp.exp(srow) + _EPS)
        sc_ref[...] = sc
        off_ref[...] = -mrow * sc

    o_ref[...] = x_ref[...] * sc_ref[...] + off_ref[...]


def kernel(x, context_id, initial_mean, initial_std):
    b, s, d = x.shape
    n_ctx = initial_mean.shape[0]
    cid = context_id[:, 0].astype(jnp.int32)

    bs = 512
    grid = (b, s // bs)
    out = pl.pallas_call(
        _tc_body,
        grid=grid,
        in_specs=[
            pl.BlockSpec(memory_space=pltpu.SMEM),
            pl.BlockSpec((n_ctx, d), lambda i, j: (0, 0)),
            pl.BlockSpec((n_ctx, d), lambda i, j: (0, 0)),
            pl.BlockSpec((1, bs, d), lambda i, j: (i, j, 0)),
        ],
        out_specs=pl.BlockSpec((1, bs, d), lambda i, j: (i, j, 0)),
        out_shape=jax.ShapeDtypeStruct((b, s, d), x.dtype),
        scratch_shapes=[
            pltpu.VMEM((1, d), jnp.float32),
            pltpu.VMEM((1, d), jnp.float32),
        ],
        compiler_params=pltpu.CompilerParams(
            dimension_semantics=("parallel", "arbitrary"),
        ),
    )(cid, initial_mean, initial_std, x)
    return out
